# baseline (speedup 1.0000x reference)
import functools

import jax
import jax.numpy as jnp
from jax import lax
from jax.experimental import pallas as pl
from jax.experimental.pallas import tpu as pltpu
from jax.experimental.pallas import tpu_sc as plsc

_N = 10000
_D = 128
_H = 64
_C = 10

_NC = 2
_NS = 16
_NW = _NC * _NS
_CW = 128
_CH0 = 132
_CH1 = 28
_CH_PAIR = _CH0 + _CH1
_CH_MAX = max(_CH0, _CH1)
_NBUF = 4
_N_ACC = 10240
_RPT = _N_ACC // _NS

_ROWS_BLK = 1000


def _edge_agg_body(src_hbm, dst_hbm, y_hbm, zeros_hbm, out_hbm,
                   src_v, dst_v, rows_v, acc_sh, sem_g, sem_s):
    c = lax.axis_index("c")
    s = lax.axis_index("s")
    pltpu.sync_copy(zeros_hbm, acc_sh.at[pl.ds(s * _RPT, _RPT)])
    plsc.subcore_barrier()

    def run(chunk_lo, n_chunks):
        pltpu.sync_copy(src_hbm.at[s, pl.ds(chunk_lo, n_chunks)],
                        src_v.at[pl.ds(0, n_chunks)])
        pltpu.sync_copy(dst_hbm.at[s, pl.ds(chunk_lo, n_chunks)],
                        dst_v.at[pl.ds(0, n_chunks)])

        def body(g, carry):
            for b in range(_NBUF):
                pltpu.async_copy(y_hbm.at[src_v.at[g * _NBUF + b]],
                                 rows_v.at[b], sem_g)
            for b in range(_NBUF):
                pltpu.make_async_copy(y_hbm.at[pl.ds(0, _CW)],
                                      rows_v.at[b], sem_g).wait()
            for b in range(_NBUF):
                pltpu.async_copy(rows_v.at[b],
                                 acc_sh.at[dst_v.at[g * _NBUF + b]], sem_s,
                                 add=True)
            for b in range(_NBUF):
                pltpu.make_async_copy(rows_v.at[b],
                                      acc_sh.at[pl.ds(0, _CW)], sem_s).wait()
            return carry

        lax.fori_loop(0, n_chunks // _NBUF, body, 0)

    @pl.when(c == 0)
    def _():
        run(0, _CH0)

    @pl.when(c == 1)
    def _():
        run(_CH0, _CH1)

    plsc.subcore_barrier()
    pltpu.sync_copy(acc_sh.at[pl.ds(s * _RPT, _RPT)],
                    out_hbm.at[c, pl.ds(s * _RPT, _RPT)])


@functools.cache
def _get_edge_agg():
    mesh = plsc.VectorSubcoreMesh(core_axis_name="c", subcore_axis_name="s",
                                  num_cores=_NC, num_subcores=_NS)
    return pl.kernel(
        _edge_agg_body,
        out_type=jax.ShapeDtypeStruct((_NC, _N_ACC, _H), jnp.float32),
        mesh=mesh,
        scratch_types=[
            pltpu.VMEM((_CH_MAX, _CW), jnp.int32),
            pltpu.VMEM((_CH_MAX, _CW), jnp.int32),
            pltpu.VMEM((_NBUF, _CW, _H), jnp.float32),
            pltpu.VMEM_SHARED((_N_ACC, _H), jnp.float32),
            pltpu.SemaphoreType.DMA,
            pltpu.SemaphoreType.DMA,
        ],
        compiler_params=pltpu.CompilerParams(use_tc_tiling_on_sc=False),
    )


def _l1_body(x_ref, wr_ref, wo_ref, y_ref, r_ref):
    xv = x_ref[...]
    y_ref[...] = jnp.dot(xv, wr_ref[...], preferred_element_type=jnp.float32)
    r_ref[...] = jnp.dot(xv, wo_ref[...], preferred_element_type=jnp.float32)


def _layer1(x, w_rel, w_root):
    return pl.pallas_call(
        _l1_body,
        grid=(_N // _ROWS_BLK,),
        in_specs=[pl.BlockSpec((_ROWS_BLK, _D), lambda i: (i, 0)),
                  pl.BlockSpec((_D, _H), lambda i: (0, 0)),
                  pl.BlockSpec((_D, _H), lambda i: (0, 0))],
        out_specs=[pl.BlockSpec((_ROWS_BLK, _H), lambda i: (i, 0)),
                   pl.BlockSpec((_ROWS_BLK, _H), lambda i: (i, 0))],
        out_shape=[jax.ShapeDtypeStruct((_N, _H), jnp.float32)] * 2,
    )(x, w_rel, w_root)


def _mid_body(agg_ref, r_ref, wr_ref, wo_ref, y_ref, rn_ref):
    h = jnp.maximum(agg_ref[0] + agg_ref[1] + r_ref[...], 0.0)
    y_ref[...] = jnp.dot(h, wr_ref[...], preferred_element_type=jnp.float32)
    rn_ref[...] = jnp.dot(h, wo_ref[...], preferred_element_type=jnp.float32)


def _mid(agg, r, w_rel, w_root):
    return pl.pallas_call(
        _mid_body,
        grid=(_N // _ROWS_BLK,),
        in_specs=[pl.BlockSpec((_NC, _ROWS_BLK, _H), lambda i: (0, i, 0)),
                  pl.BlockSpec((_ROWS_BLK, _H), lambda i: (i, 0)),
                  pl.BlockSpec((_H, _H), lambda i: (0, 0)),
                  pl.BlockSpec((_H, _H), lambda i: (0, 0))],
        out_specs=[pl.BlockSpec((_ROWS_BLK, _H), lambda i: (i, 0)),
                   pl.BlockSpec((_ROWS_BLK, _H), lambda i: (i, 0))],
        out_shape=[jax.ShapeDtypeStruct((_N, _H), jnp.float32)] * 2,
    )(agg, r, w_rel, w_root)


def _final_body(agg_ref, r_ref, b_ref, wfc_ref, bfc_ref, o_ref):
    mask = (b_ref[...] == 0).astype(jnp.float32)
    h = jnp.maximum(agg_ref[0] + agg_ref[1] + r_ref[...], 0.0)
    s = jnp.sum(h * mask, axis=0, keepdims=True)
    cnt = jnp.sum(mask)
    pooled = s / jnp.maximum(cnt, 1.0)
    logits = jnp.dot(pooled, wfc_ref[...],
                     preferred_element_type=jnp.float32) + bfc_ref[...]
    m = jnp.max(logits, axis=-1, keepdims=True)
    e = jnp.exp(logits - m)
    o_ref[...] = e / jnp.sum(e, axis=-1, keepdims=True)


def _final(agg, r, batch2, w_fc, b_fc2):
    return pl.pallas_call(
        _final_body,
        grid=(1,),
        in_specs=[pl.BlockSpec((_NC, _N, _H), lambda i: (0, 0, 0)),
                  pl.BlockSpec((_N, _H), lambda i: (0, 0)),
                  pl.BlockSpec((_N, 1), lambda i: (0, 0)),
                  pl.BlockSpec((_H, _C), lambda i: (0, 0)),
                  pl.BlockSpec((1, _C), lambda i: (0, 0))],
        out_specs=pl.BlockSpec((1, _C), lambda i: (0, 0)),
        out_shape=jax.ShapeDtypeStruct((1, _C), jnp.float32),
    )(agg, r, batch2, w_fc, b_fc2)


def kernel(x, edge_index, batch, W1_rel, W1_root, W2_rel, W2_root,
           W3_rel, W3_root, W_fc, b_fc):
    src = edge_index[0]
    dst = edge_index[1]
    e = src.shape[0]
    e_pad = _NS * _CH_PAIR * _CW
    pad = e_pad - e
    src_p = jnp.concatenate(
        [src, jnp.zeros((pad,), jnp.int32)]).reshape(_NS, _CH_PAIR, _CW)
    trash = _N + jnp.arange(pad, dtype=jnp.int32) % (_N_ACC - _N)
    dst_p = jnp.concatenate([dst, trash]).reshape(_NS, _CH_PAIR, _CW)
    zeros = jnp.zeros((_RPT, _H), jnp.float32)
    batch2 = batch.reshape(_N, 1)

    edge_agg = _get_edge_agg()
    y1, r1 = _layer1(x, W1_rel, W1_root)
    agg1 = edge_agg(src_p, dst_p, y1, zeros)
    y2, r2 = _mid(agg1, r1, W2_rel, W2_root)
    agg2 = edge_agg(src_p, dst_p, y2, zeros)
    y3, r3 = _mid(agg2, r2, W3_rel, W3_root)
    agg3 = edge_agg(src_p, dst_p, y3, zeros)
    probs = _final(agg3, r3, batch2, W_fc, b_fc.reshape(1, _C))
    return probs[0]


# speedup vs baseline: 1.1284x; 1.1284x over previous
import functools

import jax
import jax.numpy as jnp
from jax import lax
from jax.experimental import pallas as pl
from jax.experimental.pallas import tpu as pltpu
from jax.experimental.pallas import tpu_sc as plsc

_N = 10000
_D = 128
_H = 64
_C = 10

_NC = 2
_NS = 16
_NW = _NC * _NS
_CW = 128
_CH0 = 120
_CH1 = 40
_CH_PAIR = _CH0 + _CH1
_CH_MAX = max(_CH0, _CH1)
_NBUF = 4
_N_ACC = 10240
_RPT = _N_ACC // _NS

_ROWS_BLK = 1000


def _edge_agg_body(src_hbm, dst_hbm, y_hbm, zeros_hbm, out_hbm,
                   src_v, dst_v, rows_v, acc_sh, sem_g, sem_s):
    c = lax.axis_index("c")
    s = lax.axis_index("s")
    pltpu.sync_copy(zeros_hbm, acc_sh.at[pl.ds(s * _RPT, _RPT)])
    plsc.subcore_barrier()

    def run(chunk_lo, n_chunks):
        pltpu.sync_copy(src_hbm.at[s, pl.ds(chunk_lo, n_chunks)],
                        src_v.at[pl.ds(0, n_chunks)])
        pltpu.sync_copy(dst_hbm.at[s, pl.ds(chunk_lo, n_chunks)],
                        dst_v.at[pl.ds(0, n_chunks)])

        def body(g, carry):
            for b in range(_NBUF):
                pltpu.async_copy(y_hbm.at[src_v.at[g * _NBUF + b]],
                                 rows_v.at[b], sem_g)
            for b in range(_NBUF):
                pltpu.make_async_copy(y_hbm.at[pl.ds(0, _CW)],
                                      rows_v.at[b], sem_g).wait()
            for b in range(_NBUF):
                pltpu.async_copy(rows_v.at[b],
                                 acc_sh.at[dst_v.at[g * _NBUF + b]], sem_s,
                                 add=True)
            for b in range(_NBUF):
                pltpu.make_async_copy(rows_v.at[b],
                                      acc_sh.at[pl.ds(0, _CW)], sem_s).wait()
            return carry

        lax.fori_loop(0, n_chunks // _NBUF, body, 0)

    @pl.when(c == 0)
    def _():
        run(0, _CH0)

    @pl.when(c == 1)
    def _():
        run(_CH0, _CH1)

    plsc.subcore_barrier()
    pltpu.sync_copy(acc_sh.at[pl.ds(s * _RPT, _RPT)],
                    out_hbm.at[c, pl.ds(s * _RPT, _RPT)])


@functools.cache
def _get_edge_agg():
    mesh = plsc.VectorSubcoreMesh(core_axis_name="c", subcore_axis_name="s",
                                  num_cores=_NC, num_subcores=_NS)
    return pl.kernel(
        _edge_agg_body,
        out_type=jax.ShapeDtypeStruct((_NC, _N_ACC, _H), jnp.float32),
        mesh=mesh,
        scratch_types=[
            pltpu.VMEM((_CH_MAX, _CW), jnp.int32),
            pltpu.VMEM((_CH_MAX, _CW), jnp.int32),
            pltpu.VMEM((_NBUF, _CW, _H), jnp.float32),
            pltpu.VMEM_SHARED((_N_ACC, _H), jnp.float32),
            pltpu.SemaphoreType.DMA,
            pltpu.SemaphoreType.DMA,
        ],
        compiler_params=pltpu.CompilerParams(use_tc_tiling_on_sc=False),
    )


def _l1_body(x_ref, wr_ref, wo_ref, y_ref, r_ref):
    xv = x_ref[...]
    y_ref[...] = jnp.dot(xv, wr_ref[...], preferred_element_type=jnp.float32)
    r_ref[...] = jnp.dot(xv, wo_ref[...], preferred_element_type=jnp.float32)


def _layer1(x, w_rel, w_root):
    return pl.pallas_call(
        _l1_body,
        grid=(_N // _ROWS_BLK,),
        in_specs=[pl.BlockSpec((_ROWS_BLK, _D), lambda i: (i, 0)),
                  pl.BlockSpec((_D, _H), lambda i: (0, 0)),
                  pl.BlockSpec((_D, _H), lambda i: (0, 0))],
        out_specs=[pl.BlockSpec((_ROWS_BLK, _H), lambda i: (i, 0)),
                   pl.BlockSpec((_ROWS_BLK, _H), lambda i: (i, 0))],
        out_shape=[jax.ShapeDtypeStruct((_N, _H), jnp.float32)] * 2,
    )(x, w_rel, w_root)


def _mid_body(agg_ref, r_ref, wr_ref, wo_ref, y_ref, rn_ref):
    h = jnp.maximum(agg_ref[0] + agg_ref[1] + r_ref[...], 0.0)
    y_ref[...] = jnp.dot(h, wr_ref[...], preferred_element_type=jnp.float32)
    rn_ref[...] = jnp.dot(h, wo_ref[...], preferred_element_type=jnp.float32)


def _mid(agg, r, w_rel, w_root):
    return pl.pallas_call(
        _mid_body,
        grid=(_N // _ROWS_BLK,),
        in_specs=[pl.BlockSpec((_NC, _ROWS_BLK, _H), lambda i: (0, i, 0)),
                  pl.BlockSpec((_ROWS_BLK, _H), lambda i: (i, 0)),
                  pl.BlockSpec((_H, _H), lambda i: (0, 0)),
                  pl.BlockSpec((_H, _H), lambda i: (0, 0))],
        out_specs=[pl.BlockSpec((_ROWS_BLK, _H), lambda i: (i, 0)),
                   pl.BlockSpec((_ROWS_BLK, _H), lambda i: (i, 0))],
        out_shape=[jax.ShapeDtypeStruct((_N, _H), jnp.float32)] * 2,
    )(agg, r, w_rel, w_root)


def _final_body(agg_ref, r_ref, b_ref, wfc_ref, bfc_ref, o_ref):
    mask = (b_ref[...] == 0).astype(jnp.float32)
    h = jnp.maximum(agg_ref[0] + agg_ref[1] + r_ref[...], 0.0)
    s = jnp.sum(h * mask, axis=0, keepdims=True)
    cnt = jnp.sum(mask)
    pooled = s / jnp.maximum(cnt, 1.0)
    logits = jnp.dot(pooled, wfc_ref[...],
                     preferred_element_type=jnp.float32) + bfc_ref[...]
    m = jnp.max(logits, axis=-1, keepdims=True)
    e = jnp.exp(logits - m)
    o_ref[...] = e / jnp.sum(e, axis=-1, keepdims=True)


def _final(agg, r, batch2, w_fc, b_fc2):
    return pl.pallas_call(
        _final_body,
        grid=(1,),
        in_specs=[pl.BlockSpec((_NC, _N, _H), lambda i: (0, 0, 0)),
                  pl.BlockSpec((_N, _H), lambda i: (0, 0)),
                  pl.BlockSpec((_N, 1), lambda i: (0, 0)),
                  pl.BlockSpec((_H, _C), lambda i: (0, 0)),
                  pl.BlockSpec((1, _C), lambda i: (0, 0))],
        out_specs=pl.BlockSpec((1, _C), lambda i: (0, 0)),
        out_shape=jax.ShapeDtypeStruct((1, _C), jnp.float32),
    )(agg, r, batch2, w_fc, b_fc2)


def kernel(x, edge_index, batch, W1_rel, W1_root, W2_rel, W2_root,
           W3_rel, W3_root, W_fc, b_fc):
    src = edge_index[0]
    dst = edge_index[1]
    e = src.shape[0]
    e_pad = _NS * _CH_PAIR * _CW
    pad = e_pad - e
    src_p = jnp.concatenate(
        [src, jnp.zeros((pad,), jnp.int32)]).reshape(_NS, _CH_PAIR, _CW)
    trash = _N + jnp.arange(pad, dtype=jnp.int32) % (_N_ACC - _N)
    dst_p = jnp.concatenate([dst, trash]).reshape(_NS, _CH_PAIR, _CW)
    zeros = jnp.zeros((_RPT, _H), jnp.float32)
    batch2 = batch.reshape(_N, 1)

    edge_agg = _get_edge_agg()
    y1, r1 = _layer1(x, W1_rel, W1_root)
    agg1 = edge_agg(src_p, dst_p, y1, zeros)
    y2, r2 = _mid(agg1, r1, W2_rel, W2_root)
    agg2 = edge_agg(src_p, dst_p, y2, zeros)
    y3, r3 = _mid(agg2, r2, W3_rel, W3_root)
    agg3 = edge_agg(src_p, dst_p, y3, zeros)
    probs = _final(agg3, r3, batch2, W_fc, b_fc.reshape(1, _C))
    return probs[0]
